# speedup vs baseline: 4.0113x; 4.0113x over previous
import functools

import jax
import jax.numpy as jnp
from jax import lax
from jax.experimental import pallas as pl
from jax.experimental.pallas import tpu as pltpu
from jax.experimental.pallas import tpu_sc as plsc

D_EMB = 128
D_HID = 768

CHUNK = 128
NBUF = 5


def _sc_gather(idx3, table, n_workers, n_chunks):
    tokens = n_workers * n_chunks * CHUNK
    per_w = n_chunks * CHUNK
    n_groups = n_chunks // NBUF
    info = plsc.get_sparse_core_info()
    nc = info.num_cores

    mesh = plsc.VectorSubcoreMesh(core_axis_name="c", subcore_axis_name="s")

    scratch = [pltpu.VMEM((n_chunks, CHUNK), jnp.int32)]
    scratch += [pltpu.VMEM((CHUNK, D_EMB), jnp.float32) for _ in range(NBUF)]
    scratch += [pltpu.SemaphoreType.DMA for _ in range(NBUF)]

    @functools.partial(
        pl.kernel,
        mesh=mesh,
        out_type=jax.ShapeDtypeStruct((tokens, D_EMB), jnp.float32),
        scratch_types=scratch,
    )
    def gather_k(idx_hbm, table_hbm, out_hbm, idx_v, *bufs_and_sems):
        bufs = bufs_and_sems[:NBUF]
        sems = bufs_and_sems[NBUF:]
        wid = lax.axis_index("s") * nc + lax.axis_index("c")
        base_row = wid * per_w
        pltpu.sync_copy(idx_hbm.at[wid], idx_v)

        def body(g, carry):
            base_c = g * NBUF
            copies = []
            for bi in range(NBUF):
                cp = pltpu.make_async_copy(
                    table_hbm.at[idx_v.at[base_c + bi]], bufs[bi], sems[bi])
                cp.start()
                copies.append(cp)
            for bi in range(NBUF):
                copies[bi].wait()
                row0 = base_row + (base_c + bi) * CHUNK
                pltpu.sync_copy(bufs[bi], out_hbm.at[pl.ds(row0, CHUNK)])
            return carry

        lax.fori_loop(0, n_groups, body, 0)

    return gather_k(idx3, table)


def _tc_project(idx2, emb, W, b2, blk):
    tokens = emb.shape[0]
    rows = blk // D_EMB

    def mm_body(idx_ref, emb_ref, w_ref, b_ref, out_ref, mask_ref):
        acc = jnp.dot(emb_ref[...], w_ref[...],
                      preferred_element_type=jnp.float32)
        out_ref[...] = acc + b_ref[...]
        mask_ref[...] = idx_ref[...] != 0

    return pl.pallas_call(
        mm_body,
        grid=(tokens // blk,),
        in_specs=[
            pl.BlockSpec((rows, D_EMB), lambda i: (i, 0)),
            pl.BlockSpec((blk, D_EMB), lambda i: (i, 0)),
            pl.BlockSpec((D_EMB, D_HID), lambda i: (0, 0)),
            pl.BlockSpec((1, D_HID), lambda i: (0, 0)),
        ],
        out_specs=[
            pl.BlockSpec((blk, D_HID), lambda i: (i, 0)),
            pl.BlockSpec((rows, D_EMB), lambda i: (i, 0)),
        ],
        out_shape=[
            jax.ShapeDtypeStruct((tokens, D_HID), jnp.float32),
            jax.ShapeDtypeStruct((tokens // D_EMB, D_EMB), jnp.bool_),
        ],
    )(idx2, emb, W, b2)


def kernel(idxs, table, W, b):
    B, L = idxs.shape
    tokens = B * L
    info = plsc.get_sparse_core_info()
    n_workers = info.num_cores * info.num_subcores
    n_chunks = tokens // (n_workers * CHUNK)

    idx_flat = idxs.astype(jnp.int32).reshape(-1)
    idx3 = idx_flat.reshape(n_workers, n_chunks, CHUNK)
    emb = _sc_gather(idx3, table, n_workers, n_chunks)

    idx2 = idx_flat.reshape(tokens // D_EMB, D_EMB)
    hidden2, mask2 = _tc_project(idx2, emb, W, b.reshape(1, D_HID), blk=2048)
    return hidden2.reshape(B, L, D_HID), mask2.reshape(B, L)


# speedup vs baseline: 4.0289x; 1.0044x over previous
import functools

import jax
import jax.numpy as jnp
from jax import lax
from jax.experimental import pallas as pl
from jax.experimental.pallas import tpu as pltpu
from jax.experimental.pallas import tpu_sc as plsc

D_EMB = 128
D_HID = 768

CHUNK = 128
NBUF = 5
NSLICE = 5
BLK = 2048


def _sc_gather(idx3, table, n_workers, n_chunks):
    tokens = n_workers * n_chunks * CHUNK
    per_w = n_chunks * CHUNK
    n_groups = n_chunks // NBUF
    info = plsc.get_sparse_core_info()
    nc = info.num_cores

    mesh = plsc.VectorSubcoreMesh(core_axis_name="c", subcore_axis_name="s")

    scratch = [pltpu.VMEM((n_chunks, CHUNK), jnp.int32)]
    scratch += [pltpu.VMEM((CHUNK, D_EMB), jnp.float32) for _ in range(NBUF)]
    scratch += [pltpu.SemaphoreType.DMA for _ in range(NBUF)]

    @functools.partial(
        pl.kernel,
        mesh=mesh,
        out_type=jax.ShapeDtypeStruct((tokens, D_EMB), jnp.float32),
        scratch_types=scratch,
    )
    def gather_k(idx_hbm, table_hbm, out_hbm, idx_v, *bufs_and_sems):
        bufs = bufs_and_sems[:NBUF]
        sems = bufs_and_sems[NBUF:]
        wid = lax.axis_index("s") * nc + lax.axis_index("c")
        base_row = wid * per_w
        pltpu.sync_copy(idx_hbm.at[wid], idx_v)

        def body(g, carry):
            base_c = g * NBUF
            copies = []
            for bi in range(NBUF):
                cp = pltpu.make_async_copy(
                    table_hbm.at[idx_v.at[base_c + bi]], bufs[bi], sems[bi])
                cp.start()
                copies.append(cp)
            for bi in range(NBUF):
                copies[bi].wait()
                row0 = base_row + (base_c + bi) * CHUNK
                pltpu.sync_copy(bufs[bi], out_hbm.at[pl.ds(row0, CHUNK)])
            return carry

        lax.fori_loop(0, n_groups, body, 0)

    return gather_k(idx3, table)


def _tc_project_slice(emb_s, W, b2, buf, s, tokens):
    slice_tokens = emb_s.shape[0]
    g = slice_tokens // BLK
    blk0 = s * g

    def mm_body(emb_ref, w_ref, b_ref, *rest):
        out_ref = rest[-1]
        acc = jnp.dot(emb_ref[...], w_ref[...],
                      preferred_element_type=jnp.float32)
        out_ref[...] = acc + b_ref[...]

    in_specs = [
        pl.BlockSpec((BLK, D_EMB), lambda i: (i, 0)),
        pl.BlockSpec((D_EMB, D_HID), lambda i: (0, 0)),
        pl.BlockSpec((1, D_HID), lambda i: (0, 0)),
    ]
    args = [emb_s, W, b2]
    aliases = {}
    if buf is not None:
        in_specs.append(pl.BlockSpec(memory_space=pl.ANY))
        args.append(buf)
        aliases = {3: 0}

    return pl.pallas_call(
        mm_body,
        grid=(g,),
        in_specs=in_specs,
        out_specs=pl.BlockSpec((BLK, D_HID), lambda i: (blk0 + i, 0)),
        out_shape=jax.ShapeDtypeStruct((tokens, D_HID), jnp.float32),
        input_output_aliases=aliases,
    )(*args)


def _tc_mask(idx2):
    rows = idx2.shape[0]

    def mask_body(idx_ref, mask_ref):
        mask_ref[...] = idx_ref[...] != 0

    return pl.pallas_call(
        mask_body,
        grid=(8,),
        in_specs=[pl.BlockSpec((rows // 8, D_EMB), lambda i: (i, 0))],
        out_specs=pl.BlockSpec((rows // 8, D_EMB), lambda i: (i, 0)),
        out_shape=jax.ShapeDtypeStruct((rows, D_EMB), jnp.bool_),
    )(idx2)


def kernel(idxs, table, W, b):
    B, L = idxs.shape
    tokens = B * L
    info = plsc.get_sparse_core_info()
    n_workers = info.num_cores * info.num_subcores
    n_chunks = tokens // (NSLICE * n_workers * CHUNK)

    idx_flat = idxs.astype(jnp.int32).reshape(-1)
    idx4 = idx_flat.reshape(NSLICE, n_workers, n_chunks, CHUNK)
    b2 = b.reshape(1, D_HID)

    embs = [_sc_gather(idx4[s], table, n_workers, n_chunks)
            for s in range(NSLICE)]
    buf = None
    for s in range(NSLICE):
        buf = _tc_project_slice(embs[s], W, b2, buf, s, tokens)

    mask2 = _tc_mask(idx_flat.reshape(tokens // D_EMB, D_EMB))
    return buf.reshape(B, L, D_HID), mask2.reshape(B, L)


# speedup vs baseline: 4.1604x; 1.0326x over previous
import functools

import jax
import jax.numpy as jnp
from jax import lax
from jax.experimental import pallas as pl
from jax.experimental.pallas import tpu as pltpu
from jax.experimental.pallas import tpu_sc as plsc

D_EMB = 128
D_HID = 768

CHUNK = 128
NBUF = 5
NSLICE = 1
BLK = 4096


def _sc_gather(idx3, table, n_workers, n_chunks):
    tokens = n_workers * n_chunks * CHUNK
    per_w = n_chunks * CHUNK
    n_groups = n_chunks // NBUF
    info = plsc.get_sparse_core_info()
    nc = info.num_cores

    mesh = plsc.VectorSubcoreMesh(core_axis_name="c", subcore_axis_name="s")

    scratch = [pltpu.VMEM((n_chunks, CHUNK), jnp.int32)]
    scratch += [pltpu.VMEM((CHUNK, D_EMB), jnp.float32) for _ in range(NBUF)]
    scratch += [pltpu.SemaphoreType.DMA for _ in range(NBUF)]

    @functools.partial(
        pl.kernel,
        mesh=mesh,
        out_type=jax.ShapeDtypeStruct((tokens, D_EMB), jnp.float32),
        scratch_types=scratch,
    )
    def gather_k(idx_hbm, table_hbm, out_hbm, idx_v, *bufs_and_sems):
        bufs = bufs_and_sems[:NBUF]
        sems = bufs_and_sems[NBUF:]
        wid = lax.axis_index("s") * nc + lax.axis_index("c")
        base_row = wid * per_w
        pltpu.sync_copy(idx_hbm.at[wid], idx_v)

        def body(g, carry):
            base_c = g * NBUF
            copies = []
            for bi in range(NBUF):
                cp = pltpu.make_async_copy(
                    table_hbm.at[idx_v.at[base_c + bi]], bufs[bi], sems[bi])
                cp.start()
                copies.append(cp)
            for bi in range(NBUF):
                copies[bi].wait()
                row0 = base_row + (base_c + bi) * CHUNK
                pltpu.sync_copy(bufs[bi], out_hbm.at[pl.ds(row0, CHUNK)])
            return carry

        lax.fori_loop(0, n_groups, body, 0)

    return gather_k(idx3, table)


def _tc_project_slice(emb_s, W, b2, buf, s, tokens):
    slice_tokens = emb_s.shape[0]
    g = slice_tokens // BLK
    blk0 = s * g

    def mm_body(emb_ref, w_ref, b_ref, *rest):
        out_ref = rest[-1]
        acc = jnp.dot(emb_ref[...], w_ref[...],
                      preferred_element_type=jnp.float32)
        out_ref[...] = acc + b_ref[...]

    in_specs = [
        pl.BlockSpec((BLK, D_EMB), lambda i: (i, 0)),
        pl.BlockSpec((D_EMB, D_HID), lambda i: (0, 0)),
        pl.BlockSpec((1, D_HID), lambda i: (0, 0)),
    ]
    args = [emb_s, W, b2]
    aliases = {}
    if buf is not None:
        in_specs.append(pl.BlockSpec(memory_space=pl.ANY))
        args.append(buf)
        aliases = {3: 0}

    return pl.pallas_call(
        mm_body,
        grid=(g,),
        in_specs=in_specs,
        out_specs=pl.BlockSpec((BLK, D_HID), lambda i: (blk0 + i, 0)),
        out_shape=jax.ShapeDtypeStruct((tokens, D_HID), jnp.float32),
        input_output_aliases=aliases,
    )(*args)


def _tc_mask(idx2):
    rows = idx2.shape[0]

    def mask_body(idx_ref, mask_ref):
        mask_ref[...] = idx_ref[...] != 0

    return pl.pallas_call(
        mask_body,
        grid=(8,),
        in_specs=[pl.BlockSpec((rows // 8, D_EMB), lambda i: (i, 0))],
        out_specs=pl.BlockSpec((rows // 8, D_EMB), lambda i: (i, 0)),
        out_shape=jax.ShapeDtypeStruct((rows, D_EMB), jnp.bool_),
    )(idx2)


def kernel(idxs, table, W, b):
    B, L = idxs.shape
    tokens = B * L
    info = plsc.get_sparse_core_info()
    n_workers = info.num_cores * info.num_subcores
    n_chunks = tokens // (NSLICE * n_workers * CHUNK)

    idx_flat = idxs.astype(jnp.int32).reshape(-1)
    idx4 = idx_flat.reshape(NSLICE, n_workers, n_chunks, CHUNK)
    b2 = b.reshape(1, D_HID)

    embs = [_sc_gather(idx4[s], table, n_workers, n_chunks)
            for s in range(NSLICE)]
    buf = None
    for s in range(NSLICE):
        buf = _tc_project_slice(embs[s], W, b2, buf, s, tokens)

    mask2 = _tc_mask(idx_flat.reshape(tokens // D_EMB, D_EMB))
    return buf.reshape(B, L, D_HID), mask2.reshape(B, L)


# speedup vs baseline: 4.2066x; 1.0111x over previous
import functools

import jax
import jax.numpy as jnp
from jax import lax
from jax.experimental import pallas as pl
from jax.experimental.pallas import tpu as pltpu
from jax.experimental.pallas import tpu_sc as plsc

D_EMB = 128
D_HID = 768

CHUNK = 128
NBUF = 5
NSLICE = 1
BLK = 8192


def _sc_gather(idx3, table, n_workers, n_chunks):
    tokens = n_workers * n_chunks * CHUNK
    per_w = n_chunks * CHUNK
    n_groups = n_chunks // NBUF
    info = plsc.get_sparse_core_info()
    nc = info.num_cores

    mesh = plsc.VectorSubcoreMesh(core_axis_name="c", subcore_axis_name="s")

    scratch = [pltpu.VMEM((n_chunks, CHUNK), jnp.int32)]
    scratch += [pltpu.VMEM((CHUNK, D_EMB), jnp.float32) for _ in range(NBUF)]
    scratch += [pltpu.SemaphoreType.DMA for _ in range(NBUF)]

    @functools.partial(
        pl.kernel,
        mesh=mesh,
        out_type=jax.ShapeDtypeStruct((tokens, D_EMB), jnp.float32),
        scratch_types=scratch,
    )
    def gather_k(idx_hbm, table_hbm, out_hbm, idx_v, *bufs_and_sems):
        bufs = bufs_and_sems[:NBUF]
        sems = bufs_and_sems[NBUF:]
        wid = lax.axis_index("s") * nc + lax.axis_index("c")
        base_row = wid * per_w
        pltpu.sync_copy(idx_hbm.at[wid], idx_v)

        def body(g, carry):
            base_c = g * NBUF
            copies = []
            for bi in range(NBUF):
                cp = pltpu.make_async_copy(
                    table_hbm.at[idx_v.at[base_c + bi]], bufs[bi], sems[bi])
                cp.start()
                copies.append(cp)
            for bi in range(NBUF):
                copies[bi].wait()
                row0 = base_row + (base_c + bi) * CHUNK
                pltpu.sync_copy(bufs[bi], out_hbm.at[pl.ds(row0, CHUNK)])
            return carry

        lax.fori_loop(0, n_groups, body, 0)

    return gather_k(idx3, table)


def _tc_project_slice(emb_s, W, b2, buf, s, tokens):
    slice_tokens = emb_s.shape[0]
    g = slice_tokens // BLK
    blk0 = s * g

    def mm_body(emb_ref, w_ref, b_ref, *rest):
        out_ref = rest[-1]
        acc = jnp.dot(emb_ref[...], w_ref[...],
                      preferred_element_type=jnp.float32)
        out_ref[...] = acc + b_ref[...]

    in_specs = [
        pl.BlockSpec((BLK, D_EMB), lambda i: (i, 0)),
        pl.BlockSpec((D_EMB, D_HID), lambda i: (0, 0)),
        pl.BlockSpec((1, D_HID), lambda i: (0, 0)),
    ]
    args = [emb_s, W, b2]
    aliases = {}
    if buf is not None:
        in_specs.append(pl.BlockSpec(memory_space=pl.ANY))
        args.append(buf)
        aliases = {3: 0}

    return pl.pallas_call(
        mm_body,
        grid=(g,),
        in_specs=in_specs,
        out_specs=pl.BlockSpec((BLK, D_HID), lambda i: (blk0 + i, 0)),
        out_shape=jax.ShapeDtypeStruct((tokens, D_HID), jnp.float32),
        input_output_aliases=aliases,
    )(*args)


def _tc_mask(idx2):
    rows = idx2.shape[0]

    def mask_body(idx_ref, mask_ref):
        mask_ref[...] = idx_ref[...] != 0

    return pl.pallas_call(
        mask_body,
        grid=(8,),
        in_specs=[pl.BlockSpec((rows // 8, D_EMB), lambda i: (i, 0))],
        out_specs=pl.BlockSpec((rows // 8, D_EMB), lambda i: (i, 0)),
        out_shape=jax.ShapeDtypeStruct((rows, D_EMB), jnp.bool_),
    )(idx2)


def kernel(idxs, table, W, b):
    B, L = idxs.shape
    tokens = B * L
    info = plsc.get_sparse_core_info()
    n_workers = info.num_cores * info.num_subcores
    n_chunks = tokens // (NSLICE * n_workers * CHUNK)

    idx_flat = idxs.astype(jnp.int32).reshape(-1)
    idx4 = idx_flat.reshape(NSLICE, n_workers, n_chunks, CHUNK)
    b2 = b.reshape(1, D_HID)

    embs = [_sc_gather(idx4[s], table, n_workers, n_chunks)
            for s in range(NSLICE)]
    buf = None
    for s in range(NSLICE):
        buf = _tc_project_slice(embs[s], W, b2, buf, s, tokens)

    mask2 = _tc_mask(idx_flat.reshape(tokens // D_EMB, D_EMB))
    return buf.reshape(B, L, D_HID), mask2.reshape(B, L)
